# baseline (speedup 1.0000x reference)
import functools

import jax
import jax.numpy as jnp
from jax import lax
from jax.experimental import pallas as pl
from jax.experimental.pallas import tpu as pltpu
from jax.experimental.pallas import tpu_sc as plsc

N = 10000
E = 320000
D_IN = 128
D_HID = 64
D_OUT = 128

NC = 2
NS = 16
NW = NC * NS
EPT = E // NW
CHUNK = 40
NCHUNK = EPT // CHUNK
NBUF = 10
N_PAD = 10240
RPT = N_PAD // NS
DEG_W = 16

_mesh = plsc.VectorSubcoreMesh(core_axis_name="c", subcore_axis_name="s")


def _sc_agg_body(with_deg, *refs):
    (table_hbm, src_hbm, dst_hbm, z64_hbm,
     agg_out,
     src_v, dst_v, rows_v, stage_v, agg_sh, sems) = refs

    c = lax.axis_index("c")
    s = lax.axis_index("s")
    wid = c * NS + s
    row0 = s * RPT

    pltpu.sync_copy(z64_hbm.at[pl.ds(row0, RPT)], stage_v)
    pltpu.sync_copy(stage_v, agg_sh.at[pl.ds(row0, RPT)])

    pltpu.sync_copy(src_hbm.at[wid], src_v)
    pltpu.sync_copy(dst_hbm.at[wid], dst_v)
    plsc.subcore_barrier()

    for b in range(NBUF):
        pltpu.async_copy(table_hbm.at[src_v.at[b]], rows_v.at[b], sems[b])

    def body(jj, carry):
        for b in range(NBUF):
            j = jj * NBUF + b
            pltpu.make_async_copy(
                table_hbm.at[src_v.at[j]], rows_v.at[b], sems[b]).wait()
            pltpu.sync_copy(rows_v.at[b], agg_sh.at[dst_v.at[j]], add=True)
            jn = j + NBUF

            @pl.when(jn < NCHUNK)
            def _():
                pltpu.async_copy(
                    table_hbm.at[src_v.at[jn]], rows_v.at[b], sems[b])
        return carry

    lax.fori_loop(0, NCHUNK // NBUF, body, 0)
    plsc.subcore_barrier()

    pltpu.sync_copy(agg_sh.at[pl.ds(row0, RPT)], stage_v)
    pltpu.sync_copy(stage_v, agg_out.at[c, pl.ds(row0, RPT)])


def _sc_deg_body(dst_hbm, z16_hbm, ones_hbm, deg_out,
                 dst_v, ones_v, dstage_v, deg_sh):
    c = lax.axis_index("c")
    s = lax.axis_index("s")
    wid = c * NS + s
    row0 = s * RPT

    pltpu.sync_copy(z16_hbm.at[pl.ds(row0, RPT)], dstage_v)
    pltpu.sync_copy(dstage_v, deg_sh.at[pl.ds(row0, RPT)])
    pltpu.sync_copy(ones_hbm, ones_v)
    pltpu.sync_copy(dst_hbm.at[wid], dst_v)
    plsc.subcore_barrier()

    def body(j, carry):
        pltpu.sync_copy(ones_v, deg_sh.at[dst_v.at[j]], add=True)
        return carry

    lax.fori_loop(0, NCHUNK, body, 0)
    plsc.subcore_barrier()
    pltpu.sync_copy(deg_sh.at[pl.ds(row0, RPT)], dstage_v)
    pltpu.sync_copy(dstage_v, deg_out.at[c, pl.ds(row0, RPT)])


_sc_deg = pl.kernel(
    _sc_deg_body,
    out_type=jax.ShapeDtypeStruct((NC, N_PAD, DEG_W), jnp.float32),
    mesh=_mesh,
    compiler_params=pltpu.CompilerParams(use_tc_tiling_on_sc=False),
    scratch_types=[
        pltpu.VMEM((NCHUNK, CHUNK), jnp.int32),
        pltpu.VMEM((CHUNK, DEG_W), jnp.float32),
        pltpu.VMEM((RPT, DEG_W), jnp.float32),
        pltpu.VMEM_SHARED((N_PAD, DEG_W), jnp.float32),
    ],
)


_sc_agg = pl.kernel(
    functools.partial(_sc_agg_body, False),
    out_type=jax.ShapeDtypeStruct((NC, N_PAD, D_HID), jnp.float32),
    mesh=_mesh,
    compiler_params=pltpu.CompilerParams(use_tc_tiling_on_sc=False),
    scratch_types=[
        pltpu.VMEM((NCHUNK, CHUNK), jnp.int32),
        pltpu.VMEM((NCHUNK, CHUNK), jnp.int32),
        pltpu.VMEM((NBUF, CHUNK, D_HID), jnp.float32),
        pltpu.VMEM((RPT, D_HID), jnp.float32),
        pltpu.VMEM_SHARED((N_PAD, D_HID), jnp.float32),
        [pltpu.SemaphoreType.DMA] * NBUF,
    ],
)


def _tc_pre_body(x_ref, w_ref, y_ref):
    y_ref[...] = jnp.dot(x_ref[...], w_ref[...],
                         preferred_element_type=jnp.float32)


_tc_pre = pl.pallas_call(
    _tc_pre_body,
    out_shape=jax.ShapeDtypeStruct((N, D_HID), jnp.float32),
)


def _tc_mid_body(y_ref, agg_ref, deg_ref, b_ref, hemb_ref, h_ref):
    deg = deg_ref[0, 0:N, 0:1] + deg_ref[1, 0:N, 0:1] + 1.0
    total = y_ref[...] + agg_ref[0, 0:N, :] + agg_ref[1, 0:N, :]
    hemb = total / deg + b_ref[...]
    hemb_ref[...] = hemb
    h_ref[...] = jnp.maximum(hemb, 0.0)


_tc_mid = pl.pallas_call(
    _tc_mid_body,
    out_shape=(
        jax.ShapeDtypeStruct((N, D_HID), jnp.float32),
        jax.ShapeDtypeStruct((N, D_HID), jnp.float32),
    ),
)


def _tc_fin_body(h_ref, agg_ref, deg_ref, w_ref, b_ref, out_ref):
    deg = deg_ref[0, 0:N, 0:1] + deg_ref[1, 0:N, 0:1] + 1.0
    hn = (h_ref[...] + agg_ref[0, 0:N, :] + agg_ref[1, 0:N, :]) / deg
    out_ref[...] = jnp.dot(hn, w_ref[...],
                           preferred_element_type=jnp.float32) + b_ref[...]


_tc_fin = pl.pallas_call(
    _tc_fin_body,
    out_shape=jax.ShapeDtypeStruct((N, D_OUT), jnp.float32),
)


@jax.jit
def kernel(feats, edge_index, W1, b1, W2, b2):
    src3 = edge_index[0].reshape(NW, NCHUNK, CHUNK)
    dst3 = edge_index[1].reshape(NW, NCHUNK, CHUNK)
    z64 = jnp.zeros((N_PAD, D_HID), jnp.float32)
    z16 = jnp.zeros((N_PAD, DEG_W), jnp.float32)
    ones = jnp.ones((CHUNK, DEG_W), jnp.float32)

    y = _tc_pre(feats, W1)
    deg = _sc_deg(dst3, z16, ones)
    agg1 = _sc_agg(y, src3, dst3, z64)
    h_emb, h = _tc_mid(y, agg1, deg, b1.reshape(1, D_HID))
    agg2 = _sc_agg(h, src3, dst3, z64)
    h2 = _tc_fin(h, agg2, deg, W2, b2.reshape(1, D_OUT))
    return (h_emb, h2)


# speedup vs baseline: 1.0564x; 1.0564x over previous
import functools

import jax
import jax.numpy as jnp
from jax import lax
from jax.experimental import pallas as pl
from jax.experimental.pallas import tpu as pltpu
from jax.experimental.pallas import tpu_sc as plsc

N = 10000
E = 320000
D_IN = 128
D_HID = 64
D_OUT = 128

NC = 2
NS = 16
NW = NC * NS
EPT = E // NW
CHUNK = 80
NCHUNK = EPT // CHUNK
NBUF = 5
N_PAD = 10240
RPT = N_PAD // NS
DEG_W = 16

_mesh = plsc.VectorSubcoreMesh(core_axis_name="c", subcore_axis_name="s")


def _sc_agg_body(with_deg, *refs):
    (table_hbm, src_hbm, dst_hbm, z64_hbm,
     agg_out,
     src_v, dst_v, rows_v, stage_v, agg_sh, sems) = refs

    c = lax.axis_index("c")
    s = lax.axis_index("s")
    wid = c * NS + s
    row0 = s * RPT

    pltpu.sync_copy(z64_hbm.at[pl.ds(row0, RPT)], stage_v)
    pltpu.sync_copy(stage_v, agg_sh.at[pl.ds(row0, RPT)])

    pltpu.sync_copy(src_hbm.at[wid], src_v)
    pltpu.sync_copy(dst_hbm.at[wid], dst_v)
    plsc.subcore_barrier()

    for b in range(NBUF):
        pltpu.async_copy(table_hbm.at[src_v.at[b]], rows_v.at[b], sems[b])

    def body(jj, carry):
        for b in range(NBUF):
            j = jj * NBUF + b
            pltpu.make_async_copy(
                table_hbm.at[src_v.at[j]], rows_v.at[b], sems[b]).wait()
            pltpu.sync_copy(rows_v.at[b], agg_sh.at[dst_v.at[j]], add=True)
            jn = j + NBUF

            @pl.when(jn < NCHUNK)
            def _():
                pltpu.async_copy(
                    table_hbm.at[src_v.at[jn]], rows_v.at[b], sems[b])
        return carry

    lax.fori_loop(0, NCHUNK // NBUF, body, 0)
    plsc.subcore_barrier()

    pltpu.sync_copy(agg_sh.at[pl.ds(row0, RPT)], stage_v)
    pltpu.sync_copy(stage_v, agg_out.at[c, pl.ds(row0, RPT)])


def _sc_deg_body(dst_hbm, z16_hbm, ones_hbm, deg_out,
                 dst_v, ones_v, dstage_v, deg_sh):
    c = lax.axis_index("c")
    s = lax.axis_index("s")
    wid = c * NS + s
    row0 = s * RPT

    pltpu.sync_copy(z16_hbm.at[pl.ds(row0, RPT)], dstage_v)
    pltpu.sync_copy(dstage_v, deg_sh.at[pl.ds(row0, RPT)])
    pltpu.sync_copy(ones_hbm, ones_v)
    pltpu.sync_copy(dst_hbm.at[wid], dst_v)
    plsc.subcore_barrier()

    def body(j, carry):
        pltpu.sync_copy(ones_v, deg_sh.at[dst_v.at[j]], add=True)
        return carry

    lax.fori_loop(0, NCHUNK, body, 0)
    plsc.subcore_barrier()
    pltpu.sync_copy(deg_sh.at[pl.ds(row0, RPT)], dstage_v)
    pltpu.sync_copy(dstage_v, deg_out.at[c, pl.ds(row0, RPT)])


_sc_deg = pl.kernel(
    _sc_deg_body,
    out_type=jax.ShapeDtypeStruct((NC, N_PAD, DEG_W), jnp.float32),
    mesh=_mesh,
    compiler_params=pltpu.CompilerParams(use_tc_tiling_on_sc=False),
    scratch_types=[
        pltpu.VMEM((NCHUNK, CHUNK), jnp.int32),
        pltpu.VMEM((CHUNK, DEG_W), jnp.float32),
        pltpu.VMEM((RPT, DEG_W), jnp.float32),
        pltpu.VMEM_SHARED((N_PAD, DEG_W), jnp.float32),
    ],
)


_sc_agg = pl.kernel(
    functools.partial(_sc_agg_body, False),
    out_type=jax.ShapeDtypeStruct((NC, N_PAD, D_HID), jnp.float32),
    mesh=_mesh,
    compiler_params=pltpu.CompilerParams(use_tc_tiling_on_sc=False),
    scratch_types=[
        pltpu.VMEM((NCHUNK, CHUNK), jnp.int32),
        pltpu.VMEM((NCHUNK, CHUNK), jnp.int32),
        pltpu.VMEM((NBUF, CHUNK, D_HID), jnp.float32),
        pltpu.VMEM((RPT, D_HID), jnp.float32),
        pltpu.VMEM_SHARED((N_PAD, D_HID), jnp.float32),
        [pltpu.SemaphoreType.DMA] * NBUF,
    ],
)


def _tc_pre_body(x_ref, w_ref, y_ref):
    y_ref[...] = jnp.dot(x_ref[...], w_ref[...],
                         preferred_element_type=jnp.float32)


_tc_pre = pl.pallas_call(
    _tc_pre_body,
    out_shape=jax.ShapeDtypeStruct((N, D_HID), jnp.float32),
)


def _tc_mid_body(y_ref, agg_ref, deg_ref, b_ref, hemb_ref, h_ref):
    deg = deg_ref[0, 0:N, 0:1] + deg_ref[1, 0:N, 0:1] + 1.0
    total = y_ref[...] + agg_ref[0, 0:N, :] + agg_ref[1, 0:N, :]
    hemb = total / deg + b_ref[...]
    hemb_ref[...] = hemb
    h_ref[...] = jnp.maximum(hemb, 0.0)


_tc_mid = pl.pallas_call(
    _tc_mid_body,
    out_shape=(
        jax.ShapeDtypeStruct((N, D_HID), jnp.float32),
        jax.ShapeDtypeStruct((N, D_HID), jnp.float32),
    ),
)


def _tc_fin_body(h_ref, agg_ref, deg_ref, w_ref, b_ref, out_ref):
    deg = deg_ref[0, 0:N, 0:1] + deg_ref[1, 0:N, 0:1] + 1.0
    hn = (h_ref[...] + agg_ref[0, 0:N, :] + agg_ref[1, 0:N, :]) / deg
    out_ref[...] = jnp.dot(hn, w_ref[...],
                           preferred_element_type=jnp.float32) + b_ref[...]


_tc_fin = pl.pallas_call(
    _tc_fin_body,
    out_shape=jax.ShapeDtypeStruct((N, D_OUT), jnp.float32),
)


@jax.jit
def kernel(feats, edge_index, W1, b1, W2, b2):
    src3 = edge_index[0].reshape(NW, NCHUNK, CHUNK)
    dst3 = edge_index[1].reshape(NW, NCHUNK, CHUNK)
    z64 = jnp.zeros((N_PAD, D_HID), jnp.float32)
    z16 = jnp.zeros((N_PAD, DEG_W), jnp.float32)
    ones = jnp.ones((CHUNK, DEG_W), jnp.float32)

    y = _tc_pre(feats, W1)
    deg = _sc_deg(dst3, z16, ones)
    agg1 = _sc_agg(y, src3, dst3, z64)
    h_emb, h = _tc_mid(y, agg1, deg, b1.reshape(1, D_HID))
    agg2 = _sc_agg(h, src3, dst3, z64)
    h2 = _tc_fin(h, agg2, deg, W2, b2.reshape(1, D_OUT))
    return (h_emb, h2)


# speedup vs baseline: 1.0714x; 1.0142x over previous
import functools

import jax
import jax.numpy as jnp
from jax import lax
from jax.experimental import pallas as pl
from jax.experimental.pallas import tpu as pltpu
from jax.experimental.pallas import tpu_sc as plsc

N = 10000
E = 320000
D_IN = 128
D_HID = 64
D_OUT = 128

NC = 2
NS = 16
NW = NC * NS
EPT = E // NW
CHUNK = 80
NCHUNK = EPT // CHUNK
NBUF = 5
N_PAD = 10240
RPT = N_PAD // NS
DEG_W = 8

_mesh = plsc.VectorSubcoreMesh(core_axis_name="c", subcore_axis_name="s")


def _sc_agg_body(with_deg, *refs):
    (table_hbm, src_hbm, dst_hbm, z64_hbm,
     agg_out,
     src_v, dst_v, rows_v, stage_v, agg_sh, sems) = refs

    c = lax.axis_index("c")
    s = lax.axis_index("s")
    wid = c * NS + s
    row0 = s * RPT

    pltpu.sync_copy(z64_hbm.at[pl.ds(row0, RPT)], stage_v)
    pltpu.sync_copy(stage_v, agg_sh.at[pl.ds(row0, RPT)])

    pltpu.sync_copy(src_hbm.at[wid], src_v)
    pltpu.sync_copy(dst_hbm.at[wid], dst_v)
    plsc.subcore_barrier()

    for b in range(NBUF):
        pltpu.async_copy(table_hbm.at[src_v.at[b]], rows_v.at[b], sems[b])

    def body(jj, carry):
        for b in range(NBUF):
            j = jj * NBUF + b
            pltpu.make_async_copy(
                table_hbm.at[src_v.at[j]], rows_v.at[b], sems[b]).wait()
            pltpu.sync_copy(rows_v.at[b], agg_sh.at[dst_v.at[j]], add=True)
            jn = j + NBUF

            @pl.when(jn < NCHUNK)
            def _():
                pltpu.async_copy(
                    table_hbm.at[src_v.at[jn]], rows_v.at[b], sems[b])
        return carry

    lax.fori_loop(0, NCHUNK // NBUF, body, 0)
    plsc.subcore_barrier()

    pltpu.sync_copy(agg_sh.at[pl.ds(row0, RPT)], stage_v)
    pltpu.sync_copy(stage_v, agg_out.at[c, pl.ds(row0, RPT)])


def _sc_deg_body(dst_hbm, z16_hbm, ones_hbm, deg_out,
                 dst_v, ones_v, dstage_v, deg_sh):
    c = lax.axis_index("c")
    s = lax.axis_index("s")
    wid = c * NS + s
    row0 = s * RPT

    pltpu.sync_copy(z16_hbm.at[pl.ds(row0, RPT)], dstage_v)
    pltpu.sync_copy(dstage_v, deg_sh.at[pl.ds(row0, RPT)])
    pltpu.sync_copy(ones_hbm, ones_v)
    pltpu.sync_copy(dst_hbm.at[wid], dst_v)
    plsc.subcore_barrier()

    def body(j, carry):
        pltpu.sync_copy(ones_v, deg_sh.at[dst_v.at[j]], add=True)
        return carry

    lax.fori_loop(0, NCHUNK, body, 0)
    plsc.subcore_barrier()
    pltpu.sync_copy(deg_sh.at[pl.ds(row0, RPT)], dstage_v)
    pltpu.sync_copy(dstage_v, deg_out.at[c, pl.ds(row0, RPT)])


_sc_deg = pl.kernel(
    _sc_deg_body,
    out_type=jax.ShapeDtypeStruct((NC, N_PAD, DEG_W), jnp.float32),
    mesh=_mesh,
    compiler_params=pltpu.CompilerParams(use_tc_tiling_on_sc=False),
    scratch_types=[
        pltpu.VMEM((NCHUNK, CHUNK), jnp.int32),
        pltpu.VMEM((CHUNK, DEG_W), jnp.float32),
        pltpu.VMEM((RPT, DEG_W), jnp.float32),
        pltpu.VMEM_SHARED((N_PAD, DEG_W), jnp.float32),
    ],
)


_sc_agg = pl.kernel(
    functools.partial(_sc_agg_body, False),
    out_type=jax.ShapeDtypeStruct((NC, N_PAD, D_HID), jnp.float32),
    mesh=_mesh,
    compiler_params=pltpu.CompilerParams(use_tc_tiling_on_sc=False),
    scratch_types=[
        pltpu.VMEM((NCHUNK, CHUNK), jnp.int32),
        pltpu.VMEM((NCHUNK, CHUNK), jnp.int32),
        pltpu.VMEM((NBUF, CHUNK, D_HID), jnp.float32),
        pltpu.VMEM((RPT, D_HID), jnp.float32),
        pltpu.VMEM_SHARED((N_PAD, D_HID), jnp.float32),
        [pltpu.SemaphoreType.DMA] * NBUF,
    ],
)


def _tc_pre_body(x_ref, w_ref, y_ref):
    y_ref[...] = jnp.dot(x_ref[...], w_ref[...],
                         preferred_element_type=jnp.float32)


_tc_pre = pl.pallas_call(
    _tc_pre_body,
    out_shape=jax.ShapeDtypeStruct((N, D_HID), jnp.float32),
)


def _tc_mid_body(y_ref, agg_ref, deg_ref, b_ref, hemb_ref, h_ref):
    deg = deg_ref[0, 0:N, 0:1] + deg_ref[1, 0:N, 0:1] + 1.0
    total = y_ref[...] + agg_ref[0, 0:N, :] + agg_ref[1, 0:N, :]
    hemb = total / deg + b_ref[...]
    hemb_ref[...] = hemb
    h_ref[...] = jnp.maximum(hemb, 0.0)


_tc_mid = pl.pallas_call(
    _tc_mid_body,
    out_shape=(
        jax.ShapeDtypeStruct((N, D_HID), jnp.float32),
        jax.ShapeDtypeStruct((N, D_HID), jnp.float32),
    ),
)


def _tc_fin_body(h_ref, agg_ref, deg_ref, w_ref, b_ref, out_ref):
    deg = deg_ref[0, 0:N, 0:1] + deg_ref[1, 0:N, 0:1] + 1.0
    hn = (h_ref[...] + agg_ref[0, 0:N, :] + agg_ref[1, 0:N, :]) / deg
    out_ref[...] = jnp.dot(hn, w_ref[...],
                           preferred_element_type=jnp.float32) + b_ref[...]


_tc_fin = pl.pallas_call(
    _tc_fin_body,
    out_shape=jax.ShapeDtypeStruct((N, D_OUT), jnp.float32),
)


@jax.jit
def kernel(feats, edge_index, W1, b1, W2, b2):
    src3 = edge_index[0].reshape(NW, NCHUNK, CHUNK)
    dst3 = edge_index[1].reshape(NW, NCHUNK, CHUNK)
    z64 = jnp.zeros((N_PAD, D_HID), jnp.float32)
    z16 = jnp.zeros((N_PAD, DEG_W), jnp.float32)
    ones = jnp.ones((CHUNK, DEG_W), jnp.float32)

    y = _tc_pre(feats, W1)
    deg = _sc_deg(dst3, z16, ones)
    agg1 = _sc_agg(y, src3, dst3, z64)
    h_emb, h = _tc_mid(y, agg1, deg, b1.reshape(1, D_HID))
    agg2 = _sc_agg(h, src3, dst3, z64)
    h2 = _tc_fin(h, agg2, deg, W2, b2.reshape(1, D_OUT))
    return (h_emb, h2)
